# baseline (speedup 1.0000x reference)
import jax
import jax.numpy as jnp
from jax.experimental import pallas as pl

_B, _N, _F = 4, 512, 128
_BPS = 4

_DN0 = (((0,), (0,)), ((), ()))


def _agg(a, m):
    t = jax.lax.dot_general(a, m, _DN0, preferred_element_type=jnp.float32)
    return t + m


def _gcn_batch_kernel(adj_ref, x_ref, w1_ref, b1_ref, w2_ref, b2_ref, out_ref):
    xw_all = jnp.dot(x_ref[:].reshape(_BPS * _N, _F), w1_ref[:],
                     preferred_element_type=jnp.float32)
    for i in range(_BPS):
        a = adj_ref[i]
        deg = jnp.sum(a, axis=0, keepdims=True) + 1.0
        dinv = jnp.transpose(jax.lax.rsqrt(deg), (1, 0))

        xw = xw_all[i * _N:(i + 1) * _N]
        h = jnp.maximum(_agg(a, xw * dinv) * dinv + b1_ref[:], 0.0)

        hw = jnp.dot(h, w2_ref[:], preferred_element_type=jnp.float32)
        h2 = _agg(a, hw * dinv) * dinv + b2_ref[:]

        out_ref[i] = jnp.sum(h2, axis=0, keepdims=True) * (1.0 / _N)


@jax.jit
def kernel(x, adj, W1, b1, W2, b2):
    b1r = b1.reshape(1, -1)
    b2r = b2.reshape(1, -1)
    grid = (_B // _BPS,)
    return pl.pallas_call(
        _gcn_batch_kernel,
        grid=grid,
        in_specs=[
            pl.BlockSpec((_BPS, _N, _N), lambda b: (b, 0, 0)),
            pl.BlockSpec((_BPS, _N, _F), lambda b: (b, 0, 0)),
            pl.BlockSpec((_F, _F), lambda b: (0, 0)),
            pl.BlockSpec((1, _F), lambda b: (0, 0)),
            pl.BlockSpec((_F, _F), lambda b: (0, 0)),
            pl.BlockSpec((1, _F), lambda b: (0, 0)),
        ],
        out_specs=pl.BlockSpec((_BPS, 1, _F), lambda b: (b, 0, 0)),
        out_shape=jax.ShapeDtypeStruct((_B, 1, _F), jnp.float32),
    )(adj, x, W1, b1r, W2, b2r).reshape(_B, _F)
